# baseline (speedup 1.0000x reference)
import functools

import jax
import jax.numpy as jnp
from jax import lax
from jax.experimental import pallas as pl
from jax.experimental.pallas import tpu as pltpu
from jax.experimental.pallas import tpu_sc as plsc

N = 20000
G = 128
NC = 2
NS = 16
L = 16
NW = NC * NS
CHUNK = 640
NPAD = NW * CHUNK
NGROUP = CHUNK // L
GL = G * L
UNROLL = 2
UNROLL2 = 8
POS_THR = 0.5
NEG_THR = 0.4

_mesh = plsc.VectorSubcoreMesh(core_axis_name="c", subcore_axis_name="s")

_GDN = lax.GatherDimensionNumbers(
    offset_dims=(), collapsed_slice_dims=(0,), start_index_map=(0,))


def _shuf(x, idx):
    return lax.gather(x, idx[:, None], _GDN, (1,),
                      mode=lax.GatherScatterMode.PROMISE_IN_BOUNDS)


_f32 = jnp.float32
_i32 = jnp.int32


TAIL = N - (NW - 1) * CHUNK


@functools.partial(
    pl.kernel,
    out_type=[
        jax.ShapeDtypeStruct((N,), _f32),
        jax.ShapeDtypeStruct((NPAD,), _i32),
        jax.ShapeDtypeStruct((NW * G,), _f32),
        jax.ShapeDtypeStruct((NW, NGROUP, GL), _f32),
    ],
    mesh=_mesh,
    scratch_types=[
        pltpu.VMEM((CHUNK,), _f32),
        pltpu.VMEM((CHUNK,), _f32),
        pltpu.VMEM((CHUNK,), _f32),
        pltpu.VMEM((CHUNK,), _f32),
        pltpu.VMEM((G * 8 + L,), _f32),
        pltpu.VMEM((GL,), _f32),
        pltpu.VMEM((GL,), _f32),
        pltpu.VMEM((GL,), _f32),
        pltpu.VMEM((GL,), _f32),
        pltpu.VMEM((GL,), _f32),
        pltpu.VMEM((GL,), _f32),
        pltpu.VMEM((GL,), _f32),
        pltpu.VMEM((GL,), _f32),
        pltpu.VMEM((GL,), _f32),
        pltpu.VMEM((GL,), _f32),
        pltpu.VMEM((CHUNK,), _f32),
        pltpu.VMEM((CHUNK,), _i32),
        pltpu.VMEM((G,), _f32),
        pltpu.SemaphoreType.DMA,
        pltpu.SemaphoreType.DMA,
        pltpu.SemaphoreType.DMA,
        pltpu.SemaphoreType.DMA,
    ],
)
def _phase1(ax1_h, ay1_h, ax2_h, ay2_h, gt_h,
            rowmax_h, argmax_h, colpart_h, ovmat_h,
            sax1, say1, sax2, say2, stgt, sgx1, sgy1, sgx2, sgy2, sga,
            sov0, sov1, sov2, sov3, scol, srm, sam, scolred,
            sem0, sem1, sem2, sem3):
    cid = lax.axis_index("c")
    sid = lax.axis_index("s")
    w = sid * NC + cid
    base = w * CHUNK
    last = NW - 1

    pltpu.make_async_copy(gt_h, stgt.at[pl.ds(0, G * 8)], sem1).start()
    srcs = (ax1_h, ay1_h, ax2_h, ay2_h)
    dsts = (sax1, say1, sax2, say2)

    @pl.when(w < last)
    def _load_full():
        for src, dst in zip(srcs, dsts):
            pltpu.make_async_copy(src.at[pl.ds(base, CHUNK)], dst,
                                  sem0).start()
        for src, dst in zip(srcs, dsts):
            pltpu.make_async_copy(src.at[pl.ds(base, CHUNK)], dst,
                                  sem0).wait()

    @pl.when(w == last)
    def _load_tail():
        for src, dst in zip(srcs, dsts):
            pltpu.make_async_copy(src.at[pl.ds(base, TAIL)],
                                  dst.at[pl.ds(0, TAIL)], sem0).start()
        zv = jnp.zeros((L,), _f32)

        def zfill(i, _):
            o = TAIL + i * L
            sax1[pl.ds(o, L)] = zv
            say1[pl.ds(o, L)] = zv
            sax2[pl.ds(o, L)] = zv
            say2[pl.ds(o, L)] = zv
            return 0
        lax.fori_loop(0, (CHUNK - TAIL) // L, zfill, 0)
        for src, dst in zip(srcs, dsts):
            pltpu.make_async_copy(src.at[pl.ds(base, TAIL)],
                                  dst.at[pl.ds(0, TAIL)], sem0).wait()

    pltpu.make_async_copy(gt_h, stgt.at[pl.ds(0, G * 8)], sem1).wait()

    def init_j(j, _):
        jb = j * L
        v = stgt[pl.ds(j * 8, L)]
        gx1 = _shuf(v, jnp.full((L,), 0, _i32))
        gy1 = _shuf(v, jnp.full((L,), 1, _i32))
        gx2 = _shuf(v, jnp.full((L,), 2, _i32))
        gy2 = _shuf(v, jnp.full((L,), 3, _i32))
        sgx1[pl.ds(jb, L)] = gx1
        sgy1[pl.ds(jb, L)] = gy1
        sgx2[pl.ds(jb, L)] = gx2
        sgy2[pl.ds(jb, L)] = gy2
        sga[pl.ds(jb, L)] = (gx2 - gx1) * (gy2 - gy1)
        scol[pl.ds(jb, L)] = jnp.full((L,), -1.0, _f32)
        return 0
    lax.fori_loop(0, G, init_j, 0)

    def one_pair(g0, bufA, bufB):
        gbA = g0 * L
        gbB = gbA + L
        aA = [sax1[pl.ds(gbA, L)], say1[pl.ds(gbA, L)],
              sax2[pl.ds(gbA, L)], say2[pl.ds(gbA, L)]]
        aB = [sax1[pl.ds(gbB, L)], say1[pl.ds(gbB, L)],
              sax2[pl.ds(gbB, L)], say2[pl.ds(gbB, L)]]
        arA = (aA[2] - aA[0]) * (aA[3] - aA[1])
        arB = (aB[2] - aB[0]) * (aB[3] - aB[1])

        def iou_of(a, ar, gx1, gy1, gx2, gy2, ga):
            ltx = jnp.maximum(a[0], gx1)
            lty = jnp.maximum(a[1], gy1)
            rbx = jnp.minimum(a[2], gx2)
            rby = jnp.minimum(a[3], gy2)
            iw = jnp.maximum(rbx - ltx, 0.0)
            ih = jnp.maximum(rby - lty, 0.0)
            inter = iw * ih
            union = ar + ga - inter
            return inter / union

        def j_body(u, carry):
            rmA, amA, rmB, amB = carry
            j0 = u * UNROLL
            iA, iB = [], []
            for k in range(UNROLL):
                j = j0 + k
                jb = j * L
                gx1 = sgx1[pl.ds(jb, L)]
                gy1 = sgy1[pl.ds(jb, L)]
                gx2 = sgx2[pl.ds(jb, L)]
                gy2 = sgy2[pl.ds(jb, L)]
                ga = sga[pl.ds(jb, L)]
                vA = iou_of(aA, arA, gx1, gy1, gx2, gy2, ga)
                vB = iou_of(aB, arB, gx1, gy1, gx2, gy2, ga)
                bufA[pl.ds(jb, L)] = vA
                bufB[pl.ds(jb, L)] = vB
                scol[pl.ds(jb, L)] = jnp.maximum(
                    scol[pl.ds(jb, L)], jnp.maximum(vA, vB))
                iA.append(vA)
                iB.append(vB)
            def tree(vals):
                ms = list(vals)
                as_ = [j0 + k for k in range(UNROLL)]
                while len(ms) > 1:
                    nm, na = [], []
                    for p in range(0, len(ms), 2):
                        nm.append(jnp.maximum(ms[p], ms[p + 1]))
                        na.append(jnp.where(ms[p + 1] > ms[p],
                                            as_[p + 1], as_[p]))
                    ms, as_ = nm, na
                return ms[0], as_[0]

            mA, aselA = tree(iA)
            amA = jnp.where(mA > rmA, aselA, amA)
            rmA = jnp.maximum(rmA, mA)
            mB, aselB = tree(iB)
            amB = jnp.where(mB > rmB, aselB, amB)
            rmB = jnp.maximum(rmB, mB)
            return rmA, amA, rmB, amB

        z = jnp.full((L,), -1.0, _f32)
        zi = jnp.zeros((L,), _i32)
        rmA, amA, rmB, amB = lax.fori_loop(
            0, G // UNROLL, j_body, (z, zi, z, zi))
        srm[pl.ds(gbA, L)] = rmA
        sam[pl.ds(gbA, L)] = amA
        srm[pl.ds(gbB, L)] = rmB
        sam[pl.ds(gbB, L)] = amB

    halves = ((0, sov0, sem0, sov1, sem1), (1, sov2, sem2, sov3, sem3))

    def dgroup(q, _):
        for half, bA, sA, bB, sB in halves:
            g0 = (q * 2 + half) * 2

            @pl.when(q > 0)
            def _wait():
                pltpu.make_async_copy(bA, ovmat_h.at[w, g0 - 4], sA).wait()
                pltpu.make_async_copy(bB, ovmat_h.at[w, g0 - 3], sB).wait()

            one_pair(g0, bA, bB)
            pltpu.make_async_copy(bA, ovmat_h.at[w, g0], sA).start()
            pltpu.make_async_copy(bB, ovmat_h.at[w, g0 + 1], sB).start()
        return 0
    lax.fori_loop(0, NGROUP // 4, dgroup, 0)
    for i, (b, s) in enumerate(((sov0, sem0), (sov1, sem1),
                                (sov2, sem2), (sov3, sem3))):
        pltpu.make_async_copy(b, ovmat_h.at[w, NGROUP - 4 + i], s).wait()

    @pl.when(w < last)
    def _store_full():
        pltpu.make_async_copy(srm, rowmax_h.at[pl.ds(base, CHUNK)],
                              sem0).start()

    @pl.when(w == last)
    def _store_tail():
        pltpu.make_async_copy(srm.at[pl.ds(0, TAIL)],
                              rowmax_h.at[pl.ds(base, TAIL)], sem0).start()

    pltpu.make_async_copy(sam, argmax_h.at[pl.ds(w * CHUNK, CHUNK)],
                          sem1).start()

    lane = lax.iota(_i32, L)

    def red_outer(jv, _):
        def red_inner(jl, acc):
            m = scol[pl.ds((jv * L + jl) * L, L)]
            for sh in (8, 4, 2, 1):
                m = jnp.maximum(m, _shuf(m, lane ^ sh))
            return jnp.where(lane == jl, m, acc)
        acc = lax.fori_loop(0, L, red_inner, jnp.full((L,), -1.0, _f32))
        scolred[pl.ds(jv * L, L)] = acc
        return 0
    lax.fori_loop(0, G // L, red_outer, 0)

    pltpu.sync_copy(scolred, colpart_h.at[pl.ds(w * G, G)])

    @pl.when(w < last)
    def _drain_full():
        pltpu.make_async_copy(srm, rowmax_h.at[pl.ds(base, CHUNK)],
                              sem0).wait()

    @pl.when(w == last)
    def _drain_tail():
        pltpu.make_async_copy(srm.at[pl.ds(0, TAIL)],
                              rowmax_h.at[pl.ds(base, TAIL)], sem0).wait()

    pltpu.make_async_copy(sam, argmax_h.at[pl.ds(w * CHUNK, CHUNK)],
                          sem1).wait()


@functools.partial(
    pl.kernel,
    out_type=[
        jax.ShapeDtypeStruct((N,), _i32),
        jax.ShapeDtypeStruct((N,), _i32),
    ],
    mesh=_mesh,
    scratch_types=[
        pltpu.VMEM((GL,), _f32),
        pltpu.VMEM((GL,), _f32),
        pltpu.VMEM((GL,), _f32),
        pltpu.VMEM((GL,), _f32),
        pltpu.VMEM((CHUNK,), _f32),
        pltpu.VMEM((CHUNK,), _i32),
        pltpu.VMEM((NW * G,), _f32),
        pltpu.VMEM((GL,), _f32),
        pltpu.VMEM((G,), _i32),
        pltpu.VMEM((CHUNK,), _i32),
        pltpu.VMEM((CHUNK,), _i32),
        pltpu.SemaphoreType.DMA,
        pltpu.SemaphoreType.DMA,
        pltpu.SemaphoreType.DMA,
        pltpu.SemaphoreType.DMA,
        pltpu.SemaphoreType.DMA,
    ],
)
def _phase2(rowmax_h, argmax_h, colpart_h, ovmat_h, glab_h,
            assigned_h, labels_h,
            sov0, sov1, sov2, sov3, srm, sam, scp, scolb, slab, sasg, slabo,
            sem0, sem1, sem2, sem3, sem4):
    cid = lax.axis_index("c")
    sid = lax.axis_index("s")
    w = sid * NC + cid
    base = w * CHUNK

    last = NW - 1
    for i, (b, s) in enumerate(((sov0, sem0), (sov1, sem1),
                                (sov2, sem2), (sov3, sem3))):
        pltpu.make_async_copy(ovmat_h.at[w, i], b, s).start()

    pltpu.make_async_copy(argmax_h.at[pl.ds(w * CHUNK, CHUNK)], sam,
                          sem4).start()
    pltpu.make_async_copy(colpart_h, scp, sem4).start()
    pltpu.make_async_copy(glab_h, slab, sem4).start()

    @pl.when(w < last)
    def _load_full():
        pltpu.make_async_copy(rowmax_h.at[pl.ds(base, CHUNK)], srm,
                              sem4).start()
        pltpu.make_async_copy(rowmax_h.at[pl.ds(base, CHUNK)], srm,
                              sem4).wait()

    @pl.when(w == last)
    def _load_tail():
        pltpu.make_async_copy(rowmax_h.at[pl.ds(base, TAIL)],
                              srm.at[pl.ds(0, TAIL)], sem4).start()
        pltpu.make_async_copy(rowmax_h.at[pl.ds(base, TAIL)],
                              srm.at[pl.ds(0, TAIL)], sem4).wait()

    pltpu.make_async_copy(argmax_h.at[pl.ds(w * CHUNK, CHUNK)], sam,
                          sem4).wait()
    pltpu.make_async_copy(colpart_h, scp, sem4).wait()
    pltpu.make_async_copy(glab_h, slab, sem4).wait()

    for jv in range(G // L):
        acc = scp[pl.ds(jv * L, L)]
        for wi in range(1, NW):
            acc = jnp.maximum(acc, scp[pl.ds(wi * G + jv * L, L)])
        for jl in range(L):
            scolb[pl.ds((jv * L + jl) * L, L)] = _shuf(
                acc, jnp.full((L,), jl, _i32))

    labv = [slab[pl.ds(v * L, L)] for v in range(G // L)]

    def finish_group(g, lastj):
        gb = g * L
        rm = srm[pl.ds(gb, L)]
        am = sam[pl.ds(gb, L)]
        asg = jnp.where(rm > POS_THR, am + 1,
                        jnp.where(rm < NEG_THR, 0, -1))
        asg = jnp.where(lastj >= 0, lastj + 1, asg)
        safe = jnp.clip(asg - 1, 0, G - 1)
        lo = safe & (L - 1)
        hi = safe >> 4
        lbl = _shuf(labv[0], lo)
        for v in range(1, G // L):
            lbl = jnp.where(hi == v, _shuf(labv[v], lo), lbl)
        sasg[pl.ds(gb, L)] = asg
        slabo[pl.ds(gb, L)] = jnp.where(asg > 0, lbl, -1)

    def one_pair(g0, bufA, bufB):
        def j_body(u, carry):
            lastA, lastB = carry
            j0 = u * UNROLL2
            cA = jnp.full((L,), -1, _i32)
            cB = jnp.full((L,), -1, _i32)
            for k in range(UNROLL2):
                j = j0 + k
                jb = j * L
                cb = scolb[pl.ds(jb, L)]
                cA = jnp.where(bufA[pl.ds(jb, L)] == cb, j, cA)
                cB = jnp.where(bufB[pl.ds(jb, L)] == cb, j, cB)
            lastA = jnp.where(cA >= 0, cA, lastA)
            lastB = jnp.where(cB >= 0, cB, lastB)
            return lastA, lastB

        zi = jnp.full((L,), -1, _i32)
        lastA, lastB = lax.fori_loop(0, G // UNROLL2, j_body, (zi, zi))
        finish_group(g0, lastA)
        finish_group(g0 + 1, lastB)

    halves = ((0, sov0, sem0, sov1, sem1), (1, sov2, sem2, sov3, sem3))

    def dgroup(q, _):
        for half, bA, sA, bB, sB in halves:
            g0 = (q * 2 + half) * 2
            pltpu.make_async_copy(ovmat_h.at[w, g0], bA, sA).wait()
            pltpu.make_async_copy(ovmat_h.at[w, g0 + 1], bB, sB).wait()
            one_pair(g0, bA, bB)

            @pl.when(q < NGROUP // 4 - 1)
            def _prefetch():
                pltpu.make_async_copy(ovmat_h.at[w, g0 + 4], bA, sA).start()
                pltpu.make_async_copy(ovmat_h.at[w, g0 + 5], bB, sB).start()
        return 0
    lax.fori_loop(0, NGROUP // 4, dgroup, 0)

    @pl.when(w < last)
    def _store_full():
        pltpu.make_async_copy(sasg, assigned_h.at[pl.ds(base, CHUNK)],
                              sem4).start()
        pltpu.make_async_copy(slabo, labels_h.at[pl.ds(base, CHUNK)],
                              sem4).start()
        pltpu.make_async_copy(sasg, assigned_h.at[pl.ds(base, CHUNK)],
                              sem4).wait()
        pltpu.make_async_copy(slabo, labels_h.at[pl.ds(base, CHUNK)],
                              sem4).wait()

    @pl.when(w == last)
    def _store_tail():
        pltpu.make_async_copy(sasg.at[pl.ds(0, TAIL)],
                              assigned_h.at[pl.ds(base, TAIL)], sem4).start()
        pltpu.make_async_copy(slabo.at[pl.ds(0, TAIL)],
                              labels_h.at[pl.ds(base, TAIL)], sem4).start()
        pltpu.make_async_copy(sasg.at[pl.ds(0, TAIL)],
                              assigned_h.at[pl.ds(base, TAIL)], sem4).wait()
        pltpu.make_async_copy(slabo.at[pl.ds(0, TAIL)],
                              labels_h.at[pl.ds(base, TAIL)], sem4).wait()


def kernel(bboxes, targets, num_level_bboxes):
    del num_level_bboxes
    ax1, ay1, ax2, ay2 = (bboxes[:, k] for k in range(4))
    gtpad = jnp.pad(targets, ((0, 0), (0, 8 - targets.shape[1]))).reshape(-1)
    glab = targets[:, 4].astype(_i32)

    rowmax, argmax, colpart, ovmat = _phase1(ax1, ay1, ax2, ay2, gtpad)
    assigned, labels = _phase2(rowmax, argmax, colpart, ovmat, glab)
    return assigned, rowmax, labels


# speedup vs baseline: 1.0046x; 1.0046x over previous
import functools

import jax
import jax.numpy as jnp
from jax import lax
from jax.experimental import pallas as pl
from jax.experimental.pallas import tpu as pltpu
from jax.experimental.pallas import tpu_sc as plsc

N = 20000
G = 128
NC = 2
NS = 16
L = 16
NW = NC * NS
CHUNK = 640
NPAD = NW * CHUNK
NGROUP = CHUNK // L
GL = G * L
UNROLL = 2
UNROLL2 = 4
POS_THR = 0.5
NEG_THR = 0.4

_mesh = plsc.VectorSubcoreMesh(core_axis_name="c", subcore_axis_name="s")

_GDN = lax.GatherDimensionNumbers(
    offset_dims=(), collapsed_slice_dims=(0,), start_index_map=(0,))


def _shuf(x, idx):
    return lax.gather(x, idx[:, None], _GDN, (1,),
                      mode=lax.GatherScatterMode.PROMISE_IN_BOUNDS)


_f32 = jnp.float32
_i32 = jnp.int32


TAIL = N - (NW - 1) * CHUNK


@functools.partial(
    pl.kernel,
    out_type=[
        jax.ShapeDtypeStruct((N,), _f32),
        jax.ShapeDtypeStruct((NPAD,), _i32),
        jax.ShapeDtypeStruct((NW * G,), _f32),
        jax.ShapeDtypeStruct((NW, NGROUP, GL), _f32),
    ],
    mesh=_mesh,
    scratch_types=[
        pltpu.VMEM((CHUNK,), _f32),
        pltpu.VMEM((CHUNK,), _f32),
        pltpu.VMEM((CHUNK,), _f32),
        pltpu.VMEM((CHUNK,), _f32),
        pltpu.VMEM((G * 8 + L,), _f32),
        pltpu.VMEM((GL,), _f32),
        pltpu.VMEM((GL,), _f32),
        pltpu.VMEM((GL,), _f32),
        pltpu.VMEM((GL,), _f32),
        pltpu.VMEM((GL,), _f32),
        pltpu.VMEM((GL,), _f32),
        pltpu.VMEM((GL,), _f32),
        pltpu.VMEM((GL,), _f32),
        pltpu.VMEM((GL,), _f32),
        pltpu.VMEM((GL,), _f32),
        pltpu.VMEM((CHUNK,), _f32),
        pltpu.VMEM((CHUNK,), _i32),
        pltpu.VMEM((G,), _f32),
        pltpu.SemaphoreType.DMA,
        pltpu.SemaphoreType.DMA,
        pltpu.SemaphoreType.DMA,
        pltpu.SemaphoreType.DMA,
    ],
)
def _phase1(ax1_h, ay1_h, ax2_h, ay2_h, gt_h,
            rowmax_h, argmax_h, colpart_h, ovmat_h,
            sax1, say1, sax2, say2, stgt, sgx1, sgy1, sgx2, sgy2, sga,
            sov0, sov1, sov2, sov3, scol, srm, sam, scolred,
            sem0, sem1, sem2, sem3):
    cid = lax.axis_index("c")
    sid = lax.axis_index("s")
    w = sid * NC + cid
    base = w * CHUNK
    last = NW - 1

    pltpu.make_async_copy(gt_h, stgt.at[pl.ds(0, G * 8)], sem1).start()
    srcs = (ax1_h, ay1_h, ax2_h, ay2_h)
    dsts = (sax1, say1, sax2, say2)

    @pl.when(w < last)
    def _load_full():
        for src, dst in zip(srcs, dsts):
            pltpu.make_async_copy(src.at[pl.ds(base, CHUNK)], dst,
                                  sem0).start()
        for src, dst in zip(srcs, dsts):
            pltpu.make_async_copy(src.at[pl.ds(base, CHUNK)], dst,
                                  sem0).wait()

    @pl.when(w == last)
    def _load_tail():
        for src, dst in zip(srcs, dsts):
            pltpu.make_async_copy(src.at[pl.ds(base, TAIL)],
                                  dst.at[pl.ds(0, TAIL)], sem0).start()
        zv = jnp.zeros((L,), _f32)

        def zfill(i, _):
            o = TAIL + i * L
            sax1[pl.ds(o, L)] = zv
            say1[pl.ds(o, L)] = zv
            sax2[pl.ds(o, L)] = zv
            say2[pl.ds(o, L)] = zv
            return 0
        lax.fori_loop(0, (CHUNK - TAIL) // L, zfill, 0)
        for src, dst in zip(srcs, dsts):
            pltpu.make_async_copy(src.at[pl.ds(base, TAIL)],
                                  dst.at[pl.ds(0, TAIL)], sem0).wait()

    pltpu.make_async_copy(gt_h, stgt.at[pl.ds(0, G * 8)], sem1).wait()

    def init_j(j, _):
        jb = j * L
        v = stgt[pl.ds(j * 8, L)]
        gx1 = _shuf(v, jnp.full((L,), 0, _i32))
        gy1 = _shuf(v, jnp.full((L,), 1, _i32))
        gx2 = _shuf(v, jnp.full((L,), 2, _i32))
        gy2 = _shuf(v, jnp.full((L,), 3, _i32))
        sgx1[pl.ds(jb, L)] = gx1
        sgy1[pl.ds(jb, L)] = gy1
        sgx2[pl.ds(jb, L)] = gx2
        sgy2[pl.ds(jb, L)] = gy2
        sga[pl.ds(jb, L)] = (gx2 - gx1) * (gy2 - gy1)
        scol[pl.ds(jb, L)] = jnp.full((L,), -1.0, _f32)
        return 0
    lax.fori_loop(0, G, init_j, 0)

    def one_pair(g0, bufA, bufB):
        gbA = g0 * L
        gbB = gbA + L
        aA = [sax1[pl.ds(gbA, L)], say1[pl.ds(gbA, L)],
              sax2[pl.ds(gbA, L)], say2[pl.ds(gbA, L)]]
        aB = [sax1[pl.ds(gbB, L)], say1[pl.ds(gbB, L)],
              sax2[pl.ds(gbB, L)], say2[pl.ds(gbB, L)]]
        arA = (aA[2] - aA[0]) * (aA[3] - aA[1])
        arB = (aB[2] - aB[0]) * (aB[3] - aB[1])

        def iou_of(a, ar, gx1, gy1, gx2, gy2, ga):
            ltx = jnp.maximum(a[0], gx1)
            lty = jnp.maximum(a[1], gy1)
            rbx = jnp.minimum(a[2], gx2)
            rby = jnp.minimum(a[3], gy2)
            iw = jnp.maximum(rbx - ltx, 0.0)
            ih = jnp.maximum(rby - lty, 0.0)
            inter = iw * ih
            union = ar + ga - inter
            return inter / union

        def j_body(u, carry):
            rmA, amA, rmB, amB = carry
            j0 = u * UNROLL
            iA, iB = [], []
            for k in range(UNROLL):
                j = j0 + k
                jb = j * L
                gx1 = sgx1[pl.ds(jb, L)]
                gy1 = sgy1[pl.ds(jb, L)]
                gx2 = sgx2[pl.ds(jb, L)]
                gy2 = sgy2[pl.ds(jb, L)]
                ga = sga[pl.ds(jb, L)]
                vA = iou_of(aA, arA, gx1, gy1, gx2, gy2, ga)
                vB = iou_of(aB, arB, gx1, gy1, gx2, gy2, ga)
                bufA[pl.ds(jb, L)] = vA
                bufB[pl.ds(jb, L)] = vB
                scol[pl.ds(jb, L)] = jnp.maximum(
                    scol[pl.ds(jb, L)], jnp.maximum(vA, vB))
                iA.append(vA)
                iB.append(vB)
            def tree(vals):
                ms = list(vals)
                as_ = [j0 + k for k in range(UNROLL)]
                while len(ms) > 1:
                    nm, na = [], []
                    for p in range(0, len(ms), 2):
                        nm.append(jnp.maximum(ms[p], ms[p + 1]))
                        na.append(jnp.where(ms[p + 1] > ms[p],
                                            as_[p + 1], as_[p]))
                    ms, as_ = nm, na
                return ms[0], as_[0]

            mA, aselA = tree(iA)
            amA = jnp.where(mA > rmA, aselA, amA)
            rmA = jnp.maximum(rmA, mA)
            mB, aselB = tree(iB)
            amB = jnp.where(mB > rmB, aselB, amB)
            rmB = jnp.maximum(rmB, mB)
            return rmA, amA, rmB, amB

        z = jnp.full((L,), -1.0, _f32)
        zi = jnp.zeros((L,), _i32)
        rmA, amA, rmB, amB = lax.fori_loop(
            0, G // UNROLL, j_body, (z, zi, z, zi))
        srm[pl.ds(gbA, L)] = rmA
        sam[pl.ds(gbA, L)] = amA
        srm[pl.ds(gbB, L)] = rmB
        sam[pl.ds(gbB, L)] = amB

    halves = ((0, sov0, sem0, sov1, sem1), (1, sov2, sem2, sov3, sem3))

    def dgroup(q, _):
        for half, bA, sA, bB, sB in halves:
            g0 = (q * 2 + half) * 2

            @pl.when(q > 0)
            def _wait():
                pltpu.make_async_copy(bA, ovmat_h.at[w, g0 - 4], sA).wait()
                pltpu.make_async_copy(bB, ovmat_h.at[w, g0 - 3], sB).wait()

            one_pair(g0, bA, bB)
            pltpu.make_async_copy(bA, ovmat_h.at[w, g0], sA).start()
            pltpu.make_async_copy(bB, ovmat_h.at[w, g0 + 1], sB).start()
        return 0
    lax.fori_loop(0, NGROUP // 4, dgroup, 0)
    for i, (b, s) in enumerate(((sov0, sem0), (sov1, sem1),
                                (sov2, sem2), (sov3, sem3))):
        pltpu.make_async_copy(b, ovmat_h.at[w, NGROUP - 4 + i], s).wait()

    @pl.when(w < last)
    def _store_full():
        pltpu.make_async_copy(srm, rowmax_h.at[pl.ds(base, CHUNK)],
                              sem0).start()

    @pl.when(w == last)
    def _store_tail():
        pltpu.make_async_copy(srm.at[pl.ds(0, TAIL)],
                              rowmax_h.at[pl.ds(base, TAIL)], sem0).start()

    pltpu.make_async_copy(sam, argmax_h.at[pl.ds(w * CHUNK, CHUNK)],
                          sem1).start()

    lane = lax.iota(_i32, L)

    def red_outer(jv, _):
        def red_inner(jl, acc):
            m = scol[pl.ds((jv * L + jl) * L, L)]
            for sh in (8, 4, 2, 1):
                m = jnp.maximum(m, _shuf(m, lane ^ sh))
            return jnp.where(lane == jl, m, acc)
        acc = lax.fori_loop(0, L, red_inner, jnp.full((L,), -1.0, _f32))
        scolred[pl.ds(jv * L, L)] = acc
        return 0
    lax.fori_loop(0, G // L, red_outer, 0)

    pltpu.sync_copy(scolred, colpart_h.at[pl.ds(w * G, G)])

    @pl.when(w < last)
    def _drain_full():
        pltpu.make_async_copy(srm, rowmax_h.at[pl.ds(base, CHUNK)],
                              sem0).wait()

    @pl.when(w == last)
    def _drain_tail():
        pltpu.make_async_copy(srm.at[pl.ds(0, TAIL)],
                              rowmax_h.at[pl.ds(base, TAIL)], sem0).wait()

    pltpu.make_async_copy(sam, argmax_h.at[pl.ds(w * CHUNK, CHUNK)],
                          sem1).wait()


@functools.partial(
    pl.kernel,
    out_type=[
        jax.ShapeDtypeStruct((N,), _i32),
        jax.ShapeDtypeStruct((N,), _i32),
    ],
    mesh=_mesh,
    scratch_types=[
        pltpu.VMEM((GL,), _f32),
        pltpu.VMEM((GL,), _f32),
        pltpu.VMEM((GL,), _f32),
        pltpu.VMEM((GL,), _f32),
        pltpu.VMEM((CHUNK,), _f32),
        pltpu.VMEM((CHUNK,), _i32),
        pltpu.VMEM((NW * G,), _f32),
        pltpu.VMEM((GL,), _f32),
        pltpu.VMEM((G,), _i32),
        pltpu.VMEM((CHUNK,), _i32),
        pltpu.VMEM((CHUNK,), _i32),
        pltpu.SemaphoreType.DMA,
        pltpu.SemaphoreType.DMA,
        pltpu.SemaphoreType.DMA,
        pltpu.SemaphoreType.DMA,
        pltpu.SemaphoreType.DMA,
    ],
)
def _phase2(rowmax_h, argmax_h, colpart_h, ovmat_h, glab_h,
            assigned_h, labels_h,
            sov0, sov1, sov2, sov3, srm, sam, scp, scolb, slab, sasg, slabo,
            sem0, sem1, sem2, sem3, sem4):
    cid = lax.axis_index("c")
    sid = lax.axis_index("s")
    w = sid * NC + cid
    base = w * CHUNK

    last = NW - 1
    for i, (b, s) in enumerate(((sov0, sem0), (sov1, sem1),
                                (sov2, sem2), (sov3, sem3))):
        pltpu.make_async_copy(ovmat_h.at[w, i], b, s).start()

    pltpu.make_async_copy(argmax_h.at[pl.ds(w * CHUNK, CHUNK)], sam,
                          sem4).start()
    pltpu.make_async_copy(colpart_h, scp, sem4).start()
    pltpu.make_async_copy(glab_h, slab, sem4).start()

    @pl.when(w < last)
    def _load_full():
        pltpu.make_async_copy(rowmax_h.at[pl.ds(base, CHUNK)], srm,
                              sem4).start()
        pltpu.make_async_copy(rowmax_h.at[pl.ds(base, CHUNK)], srm,
                              sem4).wait()

    @pl.when(w == last)
    def _load_tail():
        pltpu.make_async_copy(rowmax_h.at[pl.ds(base, TAIL)],
                              srm.at[pl.ds(0, TAIL)], sem4).start()
        pltpu.make_async_copy(rowmax_h.at[pl.ds(base, TAIL)],
                              srm.at[pl.ds(0, TAIL)], sem4).wait()

    pltpu.make_async_copy(argmax_h.at[pl.ds(w * CHUNK, CHUNK)], sam,
                          sem4).wait()
    pltpu.make_async_copy(colpart_h, scp, sem4).wait()
    pltpu.make_async_copy(glab_h, slab, sem4).wait()

    for jv in range(G // L):
        acc = scp[pl.ds(jv * L, L)]
        for wi in range(1, NW):
            acc = jnp.maximum(acc, scp[pl.ds(wi * G + jv * L, L)])
        for jl in range(L):
            scolb[pl.ds((jv * L + jl) * L, L)] = _shuf(
                acc, jnp.full((L,), jl, _i32))

    labv = [slab[pl.ds(v * L, L)] for v in range(G // L)]

    def finish_group(g, lastj):
        gb = g * L
        rm = srm[pl.ds(gb, L)]
        am = sam[pl.ds(gb, L)]
        asg = jnp.where(rm > POS_THR, am + 1,
                        jnp.where(rm < NEG_THR, 0, -1))
        asg = jnp.where(lastj >= 0, lastj + 1, asg)
        safe = jnp.clip(asg - 1, 0, G - 1)
        lo = safe & (L - 1)
        hi = safe >> 4
        lbl = _shuf(labv[0], lo)
        for v in range(1, G // L):
            lbl = jnp.where(hi == v, _shuf(labv[v], lo), lbl)
        sasg[pl.ds(gb, L)] = asg
        slabo[pl.ds(gb, L)] = jnp.where(asg > 0, lbl, -1)

    def one_pair(g0, bufA, bufB):
        def j_body(u, carry):
            lastA, lastB = carry
            j0 = u * UNROLL2
            cA = jnp.full((L,), -1, _i32)
            cB = jnp.full((L,), -1, _i32)
            for k in range(UNROLL2):
                j = j0 + k
                jb = j * L
                cb = scolb[pl.ds(jb, L)]
                cA = jnp.where(bufA[pl.ds(jb, L)] == cb, j, cA)
                cB = jnp.where(bufB[pl.ds(jb, L)] == cb, j, cB)
            lastA = jnp.where(cA >= 0, cA, lastA)
            lastB = jnp.where(cB >= 0, cB, lastB)
            return lastA, lastB

        zi = jnp.full((L,), -1, _i32)
        lastA, lastB = lax.fori_loop(0, G // UNROLL2, j_body, (zi, zi))
        finish_group(g0, lastA)
        finish_group(g0 + 1, lastB)

    halves = ((0, sov0, sem0, sov1, sem1), (1, sov2, sem2, sov3, sem3))

    def dgroup(q, _):
        for half, bA, sA, bB, sB in halves:
            g0 = (q * 2 + half) * 2
            pltpu.make_async_copy(ovmat_h.at[w, g0], bA, sA).wait()
            pltpu.make_async_copy(ovmat_h.at[w, g0 + 1], bB, sB).wait()
            one_pair(g0, bA, bB)

            @pl.when(q < NGROUP // 4 - 1)
            def _prefetch():
                pltpu.make_async_copy(ovmat_h.at[w, g0 + 4], bA, sA).start()
                pltpu.make_async_copy(ovmat_h.at[w, g0 + 5], bB, sB).start()
        return 0
    lax.fori_loop(0, NGROUP // 4, dgroup, 0)

    @pl.when(w < last)
    def _store_full():
        pltpu.make_async_copy(sasg, assigned_h.at[pl.ds(base, CHUNK)],
                              sem4).start()
        pltpu.make_async_copy(slabo, labels_h.at[pl.ds(base, CHUNK)],
                              sem4).start()
        pltpu.make_async_copy(sasg, assigned_h.at[pl.ds(base, CHUNK)],
                              sem4).wait()
        pltpu.make_async_copy(slabo, labels_h.at[pl.ds(base, CHUNK)],
                              sem4).wait()

    @pl.when(w == last)
    def _store_tail():
        pltpu.make_async_copy(sasg.at[pl.ds(0, TAIL)],
                              assigned_h.at[pl.ds(base, TAIL)], sem4).start()
        pltpu.make_async_copy(slabo.at[pl.ds(0, TAIL)],
                              labels_h.at[pl.ds(base, TAIL)], sem4).start()
        pltpu.make_async_copy(sasg.at[pl.ds(0, TAIL)],
                              assigned_h.at[pl.ds(base, TAIL)], sem4).wait()
        pltpu.make_async_copy(slabo.at[pl.ds(0, TAIL)],
                              labels_h.at[pl.ds(base, TAIL)], sem4).wait()


def kernel(bboxes, targets, num_level_bboxes):
    del num_level_bboxes
    ax1, ay1, ax2, ay2 = (bboxes[:, k] for k in range(4))
    gtpad = jnp.pad(targets, ((0, 0), (0, 8 - targets.shape[1]))).reshape(-1)
    glab = targets[:, 4].astype(_i32)

    rowmax, argmax, colpart, ovmat = _phase1(ax1, ay1, ax2, ay2, gtpad)
    assigned, labels = _phase2(rowmax, argmax, colpart, ovmat, glab)
    return assigned, rowmax, labels
